# baseline (speedup 1.0000x reference)
import functools

import jax
import jax.numpy as jnp
from jax import lax
from jax.experimental import pallas as pl
from jax.experimental.pallas import tpu as pltpu
from jax.experimental.pallas import tpu_sc as plsc

_V = 1_000_000
_CATPAD = 2_000_896
_B = 16384
_NC = 2
_NS = 16
_L = 16
_NW = _NC * _NS
_BPW = _B // _NW

_mesh = plsc.VectorSubcoreMesh(core_axis_name="c", subcore_axis_name="s")


@functools.partial(
    pl.kernel,
    out_type=jax.ShapeDtypeStruct((_B,), jnp.float32),
    mesh=_mesh,
    scratch_types=[
        pltpu.VMEM((_BPW,), jnp.int32),
        pltpu.VMEM((_BPW,), jnp.int32),
        pltpu.VMEM((_BPW,), jnp.float32),
        pltpu.VMEM((_BPW,), jnp.float32),
        pltpu.VMEM((_BPW,), jnp.float32),
        pltpu.VMEM((_BPW,), jnp.float32),
        pltpu.SemaphoreType.DMA,
        pltpu.SemaphoreType.DMA,
    ],
)
def _loss_kernel(fi_hbm, ci_hbm, lc_hbm, tab_hbm, out_hbm,
                 fi_v, ci_v, fbv, cbv, lcv, outv, sem_f, sem_c):
    wid = lax.axis_index("s") * _NC + lax.axis_index("c")
    base = wid * _BPW
    pltpu.sync_copy(fi_hbm.at[pl.ds(base, _BPW)], fi_v)
    pltpu.sync_copy(ci_hbm.at[pl.ds(base, _BPW)], ci_v)

    @pl.loop(0, _BPW, step=_L)
    def _(c):
        s = pl.ds(c, _L)
        ci_v[s] = ci_v[s] + _V

    gf = pltpu.async_copy(tab_hbm.at[fi_v], fbv, sem_f)
    gc = pltpu.async_copy(tab_hbm.at[ci_v], cbv, sem_c)
    pltpu.sync_copy(lc_hbm.at[pl.ds(base, _BPW)], lcv)
    gf.wait()
    gc.wait()

    @pl.loop(0, _BPW, step=_L)
    def _(c):
        s = pl.ds(c, _L)
        t = fbv[s] + cbv[s] - lcv[s] - 1.0
        outv[s] = t * t

    pltpu.sync_copy(outv, out_hbm.at[pl.ds(base, _BPW)])


def kernel(focal_input, context_input, log_coocurrence_count, focal_table,
           context_table, focal_bias_table, context_bias_table):
    fi = focal_input.astype(jnp.int32)
    ci = context_input.astype(jnp.int32)
    lc = log_coocurrence_count.reshape(_B)
    cat = jnp.concatenate([focal_bias_table, context_bias_table], axis=0)
    tab = jnp.pad(cat, ((0, _CATPAD - 2 * _V), (0, 0))).reshape(_CATPAD)
    out = _loss_kernel(fi, ci, lc, tab)
    return out.reshape(_B, 1)


# speedup vs baseline: 1.9221x; 1.9221x over previous
import functools

import jax
import jax.numpy as jnp
from jax import lax
from jax.experimental import pallas as pl
from jax.experimental.pallas import tpu as pltpu
from jax.experimental.pallas import tpu_sc as plsc

_V = 1_000_000
_VPAD = 1_000_448
_B = 16384
_NC = 2
_NS = 16
_L = 16
_NW = _NC * _NS
_BPW = _B // _NW

_mesh = plsc.VectorSubcoreMesh(core_axis_name="c", subcore_axis_name="s")


@functools.partial(
    pl.kernel,
    out_type=jax.ShapeDtypeStruct((_B,), jnp.float32),
    mesh=_mesh,
    scratch_types=[
        pltpu.VMEM((_BPW,), jnp.int32),
        pltpu.VMEM((_BPW,), jnp.int32),
        pltpu.VMEM((_BPW,), jnp.float32),
        pltpu.VMEM((_BPW,), jnp.float32),
        pltpu.VMEM((_BPW,), jnp.float32),
        pltpu.VMEM((_BPW,), jnp.float32),
        pltpu.SemaphoreType.DMA,
        pltpu.SemaphoreType.DMA,
    ],
)
def _loss_kernel(fi_hbm, ci_hbm, lc_hbm, fb_hbm, cb_hbm, out_hbm,
                 fi_v, ci_v, fbv, cbv, lcv, outv, sem_f, sem_c):
    wid = lax.axis_index("s") * _NC + lax.axis_index("c")
    base = wid * _BPW
    pltpu.sync_copy(fi_hbm.at[pl.ds(base, _BPW)], fi_v)
    pltpu.sync_copy(ci_hbm.at[pl.ds(base, _BPW)], ci_v)
    gf = pltpu.async_copy(fb_hbm.at[fi_v], fbv, sem_f)
    gc = pltpu.async_copy(cb_hbm.at[ci_v], cbv, sem_c)
    pltpu.sync_copy(lc_hbm.at[pl.ds(base, _BPW)], lcv)
    gf.wait()
    gc.wait()

    @pl.loop(0, _BPW, step=_L)
    def _(c):
        s = pl.ds(c, _L)
        t = fbv[s] - 1.0 + cbv[s] - lcv[s]
        outv[s] = t * t

    pltpu.sync_copy(outv, out_hbm.at[pl.ds(base, _BPW)])


def kernel(focal_input, context_input, log_coocurrence_count, focal_table,
           context_table, focal_bias_table, context_bias_table):
    fi = focal_input.astype(jnp.int32)
    ci = context_input.astype(jnp.int32)
    lc = log_coocurrence_count.reshape(_B)
    pad = ((0, _VPAD - _V), (0, 0))
    fb = jnp.pad(focal_bias_table, pad).reshape(_VPAD)
    cb = jnp.pad(context_bias_table, pad).reshape(_VPAD)
    out = _loss_kernel(fi, ci, lc, fb, cb)
    return out.reshape(_B, 1)


# speedup vs baseline: 1.9499x; 1.0145x over previous
import functools

import jax
import jax.numpy as jnp
from jax import lax
from jax.experimental import pallas as pl
from jax.experimental.pallas import tpu as pltpu
from jax.experimental.pallas import tpu_sc as plsc

_V = 1_000_000
_VPAD = 1_000_448
_B = 16384
_NC = 2
_NS = 16
_L = 16
_NW = _NC * _NS
_BPW = _B // _NW

_mesh = plsc.VectorSubcoreMesh(core_axis_name="c", subcore_axis_name="s")


@functools.partial(
    pl.kernel,
    out_type=jax.ShapeDtypeStruct((_B,), jnp.float32),
    mesh=_mesh,
    scratch_types=[
        pltpu.VMEM((_BPW,), jnp.int32),
        pltpu.VMEM((_BPW,), jnp.int32),
        pltpu.VMEM((_BPW,), jnp.float32),
        pltpu.VMEM((_BPW,), jnp.float32),
        pltpu.VMEM((_BPW,), jnp.float32),
        pltpu.VMEM((_BPW,), jnp.float32),
        pltpu.SemaphoreType.DMA,
        pltpu.SemaphoreType.DMA,
    ],
)
def _loss_kernel(fi_hbm, ci_hbm, lc_hbm, fb_hbm, cb_hbm, out_hbm,
                 fi_v, ci_v, fbv, cbv, lcv, outv, sem_f, sem_c):
    wid = lax.axis_index("s") * _NC + lax.axis_index("c")
    base = wid * _BPW
    lf = pltpu.async_copy(fi_hbm.at[pl.ds(base, _BPW)], fi_v, sem_f)
    lc_ = pltpu.async_copy(ci_hbm.at[pl.ds(base, _BPW)], ci_v, sem_c)
    lf.wait()
    lc_.wait()
    gf = pltpu.async_copy(fb_hbm.at[fi_v], fbv, sem_f)
    gc = pltpu.async_copy(cb_hbm.at[ci_v], cbv, sem_c)
    pltpu.sync_copy(lc_hbm.at[pl.ds(base, _BPW)], lcv)
    gf.wait()
    gc.wait()

    @pl.loop(0, _BPW, step=_L)
    def _(c):
        s = pl.ds(c, _L)
        t = fbv[s] - 1.0 + cbv[s] - lcv[s]
        outv[s] = t * t

    pltpu.sync_copy(outv, out_hbm.at[pl.ds(base, _BPW)])


def kernel(focal_input, context_input, log_coocurrence_count, focal_table,
           context_table, focal_bias_table, context_bias_table):
    fi = focal_input.astype(jnp.int32)
    ci = context_input.astype(jnp.int32)
    lc = log_coocurrence_count.reshape(_B)
    pad = ((0, _VPAD - _V), (0, 0))
    fb = jnp.pad(focal_bias_table, pad).reshape(_VPAD)
    cb = jnp.pad(context_bias_table, pad).reshape(_VPAD)
    out = _loss_kernel(fi, ci, lc, fb, cb)
    return out.reshape(_B, 1)
